# baseline (speedup 1.0000x reference)
import functools

import jax
import jax.numpy as jnp
from jax.experimental import pallas as pl
from jax.experimental.pallas import tpu as pltpu

NUM_EV = 4
FETCH = NUM_EV + 1
QB = 256
KB = 4096
HB = KB // 2
PAD_ID = 2.0**25
INF = float("inf")
BIG = 1e30


def _extract5(vals, ids):
    qb = vals.shape[0]
    out_v, out_i = [], []
    for _ in range(FETCH):
        m = jnp.min(vals, axis=1, keepdims=True)
        eq = vals == m
        sel = jnp.min(jnp.where(eq, ids, PAD_ID), axis=1, keepdims=True)
        out_v.append(m)
        out_i.append(sel)
        vals = jnp.where(ids == sel, INF, vals)
    pad_v = jnp.full((qb, 8 - FETCH), INF, jnp.float32)
    pad_i = PAD_ID + jax.lax.broadcasted_iota(
        jnp.int32, (qb, 8 - FETCH), 1).astype(jnp.float32)
    return (jnp.concatenate(out_v + [pad_v], axis=1),
            jnp.concatenate(out_i + [pad_i], axis=1))


def _rounds(arr, ids_arr, nrounds, base_id):
    bvs, bis = [], []
    for r in range(nrounds):
        m = jnp.min(arr, axis=1, keepdims=True)
        eq = arr == m
        sel = jnp.min(jnp.where(eq, ids_arr, PAD_ID), axis=1, keepdims=True)
        bvs.append(m)
        bis.append(sel + base_id)
        if r != nrounds - 1:
            arr = jnp.where(ids_arr == sel, INF, arr)
    return bvs, bis


def _knn_kernel(q_ref, k_ref, dist_out, id_out, cvals, cids, *, kh, k_real):
    qi = pl.program_id(0)
    ki = pl.program_id(1)
    row0 = qi * QB

    @pl.when(ki == 0)
    def _init():
        cvals[...] = jnp.full((QB, 8), INF, jnp.float32)
        cids[...] = PAD_ID + 8 + jax.lax.broadcasted_iota(
            jnp.int32, (QB, 8), 1).astype(jnp.float32)

    q = q_ref[...]
    k = k_ref[...]
    q_sq = jnp.sum(q * q, axis=1, keepdims=True)
    k_sq = jnp.sum(k * k, axis=1)[None, :]
    kcol = jax.lax.broadcasted_iota(jnp.int32, (1, KB), 1) + ki * KB
    k_sq = jnp.where(kcol < k_real, k_sq, BIG)
    dot = jax.lax.dot_general(q, k, (((1,), (1,)), ((), ())),
                              preferred_element_type=jnp.float32)
    d = (q_sq + k_sq) - 2.0 * dot

    qb = d.shape[0]
    nc = d.shape[1] // 128
    iota128 = jax.lax.broadcasted_iota(
        jnp.int32, (qb, 128), 1).astype(jnp.float32)
    base_id = (ki * KB).astype(jnp.float32)
    cand_v = [cvals[...]]
    cand_i = [cids[...]]
    for c in range(nc):
        v = d[:, c * 128:(c + 1) * 128]
        for r in range(FETCH):
            m = jnp.min(v, axis=1, keepdims=True)
            eq = v == m
            sel = jnp.min(jnp.where(eq, iota128, PAD_ID), axis=1,
                          keepdims=True)
            cand_v.append(m)
            cand_i.append(sel + (base_id + float(c * 128)))
            if r != FETCH - 1:
                v = jnp.where(iota128 == sel, INF, v)

    mv = jnp.concatenate(cand_v, axis=1)
    mi = jnp.concatenate(cand_i, axis=1)
    nv, ni = _extract5(mv, mi)
    cvals[...] = nv
    cids[...] = ni

    @pl.when(ki == kh - 1)
    def _finalize():
        vals5 = nv[:, :FETCH]
        ids5 = ni[:, :FETCH].astype(jnp.int32)
        rows = row0 + jax.lax.broadcasted_iota(jnp.int32, (QB, 1), 0)
        match = ids5 == rows
        iota5 = jax.lax.broadcasted_iota(jnp.int32, (QB, FETCH), 1)
        pos = jnp.min(jnp.where(match, iota5, FETCH - 1), axis=1,
                      keepdims=True)
        keep_lo = jax.lax.broadcasted_iota(jnp.int32, (QB, NUM_EV), 1) < pos
        dist_out[...] = jnp.where(keep_lo, vals5[:, :NUM_EV], vals5[:, 1:])
        id_out[...] = jnp.where(keep_lo, ids5[:, :NUM_EV], ids5[:, 1:])


@jax.jit
def kernel(queries, keys):
    q, d_dim = queries.shape
    k_real = keys.shape[0]
    kh = pl.cdiv(k_real, KB)
    k_pad = kh * KB
    keys_p = jnp.pad(keys, ((0, k_pad - k_real), (0, 0)))
    qh = q // QB

    dists, ids = pl.pallas_call(
        functools.partial(_knn_kernel, kh=kh, k_real=k_real),
        grid=(qh, kh),
        in_specs=[
            pl.BlockSpec((QB, d_dim), lambda qi, ki: (qi, 0)),
            pl.BlockSpec((KB, d_dim), lambda qi, ki: (ki, 0)),
        ],
        out_specs=[
            pl.BlockSpec((QB, NUM_EV), lambda qi, ki: (qi, 0)),
            pl.BlockSpec((QB, NUM_EV), lambda qi, ki: (qi, 0)),
        ],
        out_shape=[
            jax.ShapeDtypeStruct((q, NUM_EV), jnp.float32),
            jax.ShapeDtypeStruct((q, NUM_EV), jnp.int32),
        ],
        scratch_shapes=[
            pltpu.VMEM((QB, 8), jnp.float32),
            pltpu.VMEM((QB, 8), jnp.float32),
        ],
        compiler_params=pltpu.CompilerParams(
            dimension_semantics=("parallel", "arbitrary")),
    )(queries, keys_p)
    return dists, ids


# speedup vs baseline: 6.2697x; 6.2697x over previous
import functools

import jax
import jax.numpy as jnp
from jax.experimental import pallas as pl
from jax.experimental.pallas import tpu as pltpu

NUM_EV = 4
FETCH = NUM_EV + 1
QB = 1024
KB = 4096
HB = KB // 2
PAD_ID = 2.0**25
INF = float("inf")
BIG = 1e30


def _extract5(vals, ids):
    qb = vals.shape[0]
    out_v, out_i = [], []
    for _ in range(FETCH):
        m = jnp.min(vals, axis=1, keepdims=True)
        eq = vals == m
        sel = jnp.min(jnp.where(eq, ids, PAD_ID), axis=1, keepdims=True)
        out_v.append(m)
        out_i.append(sel)
        vals = jnp.where(ids == sel, INF, vals)
    pad_v = jnp.full((qb, 8 - FETCH), INF, jnp.float32)
    pad_i = PAD_ID + jax.lax.broadcasted_iota(
        jnp.int32, (qb, 8 - FETCH), 1).astype(jnp.float32)
    return (jnp.concatenate(out_v + [pad_v], axis=1),
            jnp.concatenate(out_i + [pad_i], axis=1))


def _rounds(arr, ids_arr, nrounds, base_id):
    bvs, bis = [], []
    for r in range(nrounds):
        m = jnp.min(arr, axis=1, keepdims=True)
        eq = arr == m
        sel = jnp.min(jnp.where(eq, ids_arr, PAD_ID), axis=1, keepdims=True)
        bvs.append(m)
        bis.append(sel + base_id)
        if r != nrounds - 1:
            arr = jnp.where(ids_arr == sel, INF, arr)
    return bvs, bis


def _knn_kernel(q_ref, k_ref, dist_out, id_out, cvals, cids, *, kh, k_real):
    qi = pl.program_id(0)
    ki = pl.program_id(1)
    row0 = qi * QB

    @pl.when(ki == 0)
    def _init():
        cvals[...] = jnp.full((QB, 8), INF, jnp.float32)
        cids[...] = PAD_ID + 8 + jax.lax.broadcasted_iota(
            jnp.int32, (QB, 8), 1).astype(jnp.float32)

    q = q_ref[...]
    k = k_ref[...]
    q_sq = jnp.sum(q * q, axis=1, keepdims=True)
    k_sq = jnp.sum(k * k, axis=1)[None, :]
    kcol = jax.lax.broadcasted_iota(jnp.int32, (1, KB), 1) + ki * KB
    k_sq = jnp.where(kcol < k_real, k_sq, BIG)
    dot = jax.lax.dot_general(q, k, (((1,), (1,)), ((), ())),
                              preferred_element_type=jnp.float32)
    d = (q_sq + k_sq) - 2.0 * dot

    a, b = d[:, :HB], d[:, HB:]
    c = a <= b
    lo, hi = jnp.minimum(a, b), jnp.maximum(a, b)
    iota = jax.lax.broadcasted_iota(jnp.int32, (QB, HB), 1).astype(jnp.float32)
    il = jnp.where(c, iota, iota + float(HB))
    ih = (2.0 * iota + float(HB)) - il

    base_id = (ki * KB).astype(jnp.float32)
    lv, li = _rounds(lo, il, FETCH, base_id)
    hv, hi_ids = _rounds(hi, ih, 2, base_id)

    mv = jnp.concatenate([cvals[...]] + lv + hv, axis=1)
    mi = jnp.concatenate([cids[...]] + li + hi_ids, axis=1)
    nv, ni = _extract5(mv, mi)
    cvals[...] = nv
    cids[...] = ni

    @pl.when(ki == kh - 1)
    def _finalize():
        vals5 = nv[:, :FETCH]
        ids5 = ni[:, :FETCH].astype(jnp.int32)
        rows = row0 + jax.lax.broadcasted_iota(jnp.int32, (QB, 1), 0)
        match = ids5 == rows
        iota5 = jax.lax.broadcasted_iota(jnp.int32, (QB, FETCH), 1)
        pos = jnp.min(jnp.where(match, iota5, FETCH - 1), axis=1,
                      keepdims=True)
        keep_lo = jax.lax.broadcasted_iota(jnp.int32, (QB, NUM_EV), 1) < pos
        dist_out[...] = jnp.where(keep_lo, vals5[:, :NUM_EV], vals5[:, 1:])
        id_out[...] = jnp.where(keep_lo, ids5[:, :NUM_EV], ids5[:, 1:])


@jax.jit
def kernel(queries, keys):
    q, d_dim = queries.shape
    k_real = keys.shape[0]
    kh = pl.cdiv(k_real, KB)
    k_pad = kh * KB
    keys_p = jnp.pad(keys, ((0, k_pad - k_real), (0, 0)))
    qh = q // QB

    dists, ids = pl.pallas_call(
        functools.partial(_knn_kernel, kh=kh, k_real=k_real),
        grid=(qh, kh),
        in_specs=[
            pl.BlockSpec((QB, d_dim), lambda qi, ki: (qi, 0)),
            pl.BlockSpec((KB, d_dim), lambda qi, ki: (ki, 0)),
        ],
        out_specs=[
            pl.BlockSpec((QB, NUM_EV), lambda qi, ki: (qi, 0)),
            pl.BlockSpec((QB, NUM_EV), lambda qi, ki: (qi, 0)),
        ],
        out_shape=[
            jax.ShapeDtypeStruct((q, NUM_EV), jnp.float32),
            jax.ShapeDtypeStruct((q, NUM_EV), jnp.int32),
        ],
        scratch_shapes=[
            pltpu.VMEM((QB, 8), jnp.float32),
            pltpu.VMEM((QB, 8), jnp.float32),
        ],
        compiler_params=pltpu.CompilerParams(
            dimension_semantics=("parallel", "arbitrary")),
    )(queries, keys_p)
    return dists, ids
